# baseline (speedup 1.0000x reference)
import functools

import jax
import jax.numpy as jnp
import numpy as np
from jax import lax
from jax.experimental import pallas as pl
from jax.experimental.pallas import tpu as pltpu
from jax.experimental.pallas import tpu_sc as plsc

N = 10000
E = 160000
D = 128

NC = 2
NS = 16
NW = NC * NS

E_PAD = 163840
GTOT = (2 * E_PAD) // 128
GC0 = 80
GC1 = GTOT // NS - GC0
PW = 16
TCHUNKS = E_PAD // (NS * 128)
ROWS_PER_TILE = 640
N_ACC = NS * ROWS_PER_TILE

EB = 512
NB = 400

_SR3 = float(np.sqrt(3.0))

@functools.lru_cache(maxsize=1)
def _sc_kernels():
    mesh = plsc.VectorSubcoreMesh(core_axis_name="c", subcore_axis_name="s")

    @functools.partial(
        pl.kernel,
        out_type=(jax.ShapeDtypeStruct((2 * E_PAD, D // 2), jnp.int32),
                  jax.ShapeDtypeStruct((2 * E_PAD, PW), jnp.float32)),
        mesh=mesh,
        compiler_params=pltpu.CompilerParams(use_tc_tiling_on_sc=False),
        scratch_types=[
            pltpu.VMEM((GC0, 128), jnp.int32),
            pltpu.VMEM((128, D // 2), jnp.int32),
            pltpu.VMEM((128, D // 2), jnp.int32),
            pltpu.VMEM((128, D // 2), jnp.int32),
            pltpu.VMEM((128, D // 2), jnp.int32),
            pltpu.VMEM((128, PW), jnp.float32),
            pltpu.VMEM((128, PW), jnp.float32),
            pltpu.VMEM((128, PW), jnp.float32),
            pltpu.VMEM((128, PW), jnp.float32),
            pltpu.SemaphoreType.DMA,
            pltpu.SemaphoreType.DMA,
            pltpu.SemaphoreType.DMA,
            pltpu.SemaphoreType.DMA,
            pltpu.SemaphoreType.DMA,
            pltpu.SemaphoreType.DMA,
            pltpu.SemaphoreType.DMA,
            pltpu.SemaphoreType.DMA,
        ],
    )
    def _sc_gather(feat_hbm, pos_hbm, idx_hbm, outf_hbm, outp_hbm, idx_v,
                   f0, f1, f2, f3, p0, p1, p2, p3,
                   s0, s1, s2, s3, w0, w1, w2, w3):
        fbufs = (f0, f1, f2, f3)
        pbufs = (p0, p1, p2, p3)
        sems = (s0, s1, s2, s3)
        wsems = (w0, w1, w2, w3)
        nbuf = 4
        cid = lax.axis_index("c")
        sid = lax.axis_index("s")

        def issue(j, b):
            pltpu.async_copy(feat_hbm.at[idx_v.at[j]], fbufs[b], sems[b])
            pltpu.async_copy(pos_hbm.at[idx_v.at[j]], pbufs[b], sems[b])

        def drain(b, semarr):
            pltpu.make_async_copy(
                feat_hbm.at[pl.ds(0, 128)], fbufs[b], semarr[b]).wait()
            pltpu.make_async_copy(
                pos_hbm.at[pl.ds(0, 128)], pbufs[b], semarr[b]).wait()

        def run(chunk0, nchunks):
            base = chunk0 * 128
            pltpu.sync_copy(idx_hbm.at[pl.ds(chunk0, nchunks)],
                            idx_v.at[pl.ds(0, nchunks)])
            for b in range(nbuf - 1):
                issue(b, b)

            def outer(g, _):
                for b in range(nbuf):
                    j = g * nbuf + b
                    pb = (b + 3) % nbuf
                    pj = j + 3

                    @pl.when(pj < nchunks)
                    def _():
                        @pl.when(j >= 1)
                        def _():
                            drain(pb, wsems)

                        issue(pj, pb)

                    drain(b, sems)
                    pltpu.async_copy(
                        fbufs[b], outf_hbm.at[pl.ds(base + j * 128, 128)],
                        wsems[b])
                    pltpu.async_copy(
                        pbufs[b], outp_hbm.at[pl.ds(base + j * 128, 128)],
                        wsems[b])

                return 0

            lax.fori_loop(0, nchunks // nbuf, outer, 0)
            for b in range(nbuf):
                drain(b, wsems)

        @pl.when(cid == 0)
        def _():
            run(sid * GC0, GC0)

        @pl.when(cid == 1)
        def _():
            run(NS * GC0 + sid * GC1, GC1)

    @functools.partial(
        pl.kernel,
        out_type=(jax.ShapeDtypeStruct((N_ACC, D), jnp.float32),
                  jax.ShapeDtypeStruct((N_ACC, D), jnp.float32)),
        mesh=mesh,
        scratch_types=[
            pltpu.VMEM((TCHUNKS, 128), jnp.int32),
            pltpu.VMEM((128, D), jnp.float32),
            pltpu.VMEM((128, D), jnp.float32),
            pltpu.VMEM_SHARED((N_ACC, D), jnp.float32),
            pltpu.SemaphoreType.DMA,
            pltpu.SemaphoreType.DMA,
        ],
    )
    def _sc_scatter(em_hbm, ea_hbm, sidx_hbm, z_hbm,
                    outm_hbm, outa_hbm, sidx_v, mbuf, mbuf2, acc, sem, sem2):
        cid = lax.axis_index("c")
        sid = lax.axis_index("s")
        row0 = sid * ROWS_PER_TILE

        pltpu.sync_copy(z_hbm, mbuf)

        def zbody(t, _):
            pltpu.sync_copy(mbuf, acc.at[pl.ds(row0 + t * 128, 128)])
            return 0

        lax.fori_loop(0, ROWS_PER_TILE // 128, zbody, 0)
        pltpu.sync_copy(sidx_hbm.at[sid], sidx_v)
        plsc.subcore_barrier()

        base = sid * (TCHUNKS * 128)

        def scatter_all(data_hbm):
            pltpu.async_copy(data_hbm.at[pl.ds(base, 128)], mbuf, sem)

            def body(g, _):
                for b in range(2):
                    j = 2 * g + b
                    buf, sm = (mbuf, sem) if b == 0 else (mbuf2, sem2)
                    nbuf, nsm = (mbuf2, sem2) if b == 0 else (mbuf, sem)
                    pltpu.make_async_copy(
                        data_hbm.at[pl.ds(0, 128)], buf, sm).wait()

                    @pl.when(j + 1 < TCHUNKS)
                    def _():
                        pltpu.async_copy(
                            data_hbm.at[pl.ds(base + (j + 1) * 128, 128)],
                            nbuf, nsm)

                    pltpu.sync_copy(buf, acc.at[sidx_v.at[j]], add=True)
                return 0

            lax.fori_loop(0, TCHUNKS // 2, body, 0)

        @pl.when(cid == 0)
        def _():
            scatter_all(em_hbm)

        @pl.when(cid == 1)
        def _():
            scatter_all(ea_hbm)

        plsc.subcore_barrier()

        def copy_out(out_hbm):
            def obody(t, _):
                pltpu.sync_copy(acc.at[pl.ds(row0 + t * 128, 128)], mbuf)
                pltpu.sync_copy(mbuf, out_hbm.at[pl.ds(row0 + t * 128, 128)])
                return 0
            lax.fori_loop(0, ROWS_PER_TILE // 128, obody, 0)

        @pl.when(cid == 0)
        def _():
            copy_out(outm_hbm)

        @pl.when(cid == 1)
        def _():
            copy_out(outa_hbm)

    return _sc_gather, _sc_scatter


def _edge_body(xs_ref, xr_ref, ps_ref, pr_ref, w1s, w1r, w1ps, w1pr, w1a,
               wy1s, wy1r, w2m, w2p, w2a, wy2, outm_ref, outa_ref):
    f32 = jnp.float32
    xs = xs_ref[...]
    xr = xr_ref[...]
    r = ps_ref[...][:, :3] - pr_ref[...][:, :3]
    d = jnp.sqrt(jnp.sum(r * r, axis=-1, keepdims=True))
    rh = (r / (d + 1e-8)) * _SR3
    a1, a2, a3 = rh[:, 0:1], rh[:, 1:2], rh[:, 2:3]

    def dot(x, w):
        return jax.lax.dot_general(x.astype(jnp.bfloat16), w[...],
                                   (((1,), (0,)), ((), ())),
                                   preferred_element_type=f32)

    y1s = wy1s[0:1] + a1 * wy1s[1:2] + a2 * wy1s[2:3] + a3 * wy1s[3:4]
    y1r = wy1r[0:1] + a1 * wy1r[1:2] + a2 * wy1r[2:3] + a3 * wy1r[3:4]
    pre = (dot(xs, w1s) + dot(xr, w1r) + dot(xs * y1s, w1ps) + dot(xr * y1r, w1pr)
           + w1a[0:1] + a1 * w1a[1:2] + a2 * w1a[2:3] + a3 * w1a[3:4])
    m = pre[:, 128:] * jax.nn.sigmoid(pre[:, :128])
    y2 = wy2[0:1] + a1 * wy2[1:2] + a2 * wy2[2:3] + a3 * wy2[3:4]
    pre2 = (dot(m, w2m) + dot(m * y2, w2p)
            + w2a[0:1] + a1 * w2a[1:2] + a2 * w2a[2:3] + a3 * w2a[3:4])
    m2 = pre2[:, 128:] * jax.nn.sigmoid(pre2[:, :128])
    ones = jnp.ones_like(a1)
    zeros = jnp.zeros((xs.shape[0], D - 4), f32)
    outm_ref[...] = m2
    outa_ref[...] = jnp.concatenate([a1, a2, a3, ones, zeros], axis=-1)


def _edge_call(gfeat, gpos, w):
    nblk = E_PAD // EB
    full = lambda arr: pl.BlockSpec(arr.shape, lambda i: (0,) * arr.ndim)
    return pl.pallas_call(
        _edge_body,
        grid=(nblk,),
        in_specs=[pl.BlockSpec((EB, D), lambda i: (i, 0)),
                  pl.BlockSpec((EB, D), lambda i: (i + nblk, 0)),
                  pl.BlockSpec((EB, PW), lambda i: (i, 0)),
                  pl.BlockSpec((EB, PW), lambda i: (i + nblk, 0))]
                 + [full(a) for a in w],
        out_specs=[pl.BlockSpec((EB, D), lambda i: (i, 0)),
                   pl.BlockSpec((EB, D), lambda i: (i, 0))],
        out_shape=[jax.ShapeDtypeStruct((E_PAD, D), jnp.float32),
                   jax.ShapeDtypeStruct((E_PAD, D), jnp.float32)],
    )(gfeat, gfeat, gpos, gpos, *w)


def _node_body(x_ref, pm_ref, pa_ref,
               wy1m, wy1a, wn1x, wn1m, wn1p, wn1a, b1,
               wy2m, wy2a, wn2x, wn2m, wn2p, wn2a, b2,
               wlin, blin, out_ref):
    f32 = jnp.float32
    x = x_ref[...]
    msum = pm_ref[...]
    asum = pa_ref[...]
    deg = asum[:, 3:4]
    inv = 1.0 / jnp.maximum(deg, 1.0)
    m_i = msum * inv
    ai0 = deg * inv
    ai1 = asum[:, 0:1] * inv
    ai2 = asum[:, 1:2] * inv
    ai3 = asum[:, 2:3] * inv

    def dot(a, w):
        return jax.lax.dot_general(a.astype(jnp.bfloat16), w[...],
                                   (((1,), (0,)), ((), ())),
                                   preferred_element_type=f32)

    def apart(wa):
        return (ai0 * wa[0:1] + ai1 * wa[1:2] + ai2 * wa[2:3] + ai3 * wa[3:4])

    y1 = dot(m_i, wy1m) + apart(wy1a)
    p1_ = (dot(x, wn1x) + dot(m_i, wn1m) + dot(x * y1, wn1p)
           + apart(wn1a) + b1[0:1])
    x1 = p1_[:, 128:] * jax.nn.sigmoid(p1_[:, :128])
    y2 = dot(m_i, wy2m) + apart(wy2a)
    p2_ = (dot(x1, wn2x) + dot(m_i, wn2m) + dot(x1 * y2, wn2p)
           + apart(wn2a) + b2[0:1])
    x2 = p2_[:, 128:] * jax.nn.sigmoid(p2_[:, :128])
    out_ref[...] = dot(x2, wlin) + blin[0:1]


def _node_call(nodes, parts_m, parts_a, w):
    nblk = N // NB
    full = lambda arr: pl.BlockSpec(arr.shape, lambda i: (0,) * arr.ndim)
    return pl.pallas_call(
        _node_body,
        grid=(nblk,),
        in_specs=[pl.BlockSpec((NB, D), lambda i: (i, 0)),
                  pl.BlockSpec((NB, D), lambda i: (i, 0)),
                  pl.BlockSpec((NB, D), lambda i: (i, 0))]
                 + [full(a) for a in w],
        out_specs=pl.BlockSpec((NB, D), lambda i: (i, 0)),
        out_shape=jax.ShapeDtypeStruct((N, D), jnp.float32),
    )(nodes, parts_m, parts_a, *w)


def _pad8(w):
    return jnp.concatenate([w, jnp.zeros((8 - w.shape[0],) + w.shape[1:], w.dtype)])


def kernel(x, edge_index, Wy_e1, W_e1, b_e1, Wy_e2, W_e2, b_e2,
           Wy_n1, W_n1, b_n1, Wy_n2, W_n2, b_n2, W_lin, b_lin):
    senders = edge_index[0].astype(jnp.int32)
    receivers = edge_index[1].astype(jnp.int32)
    pad = E_PAD - E
    spad = jnp.concatenate([senders, jnp.zeros((pad,), jnp.int32)])
    rpad = jnp.concatenate([receivers, jnp.zeros((pad,), jnp.int32)])
    gidx = jnp.concatenate([spad, rpad]).reshape(GTOT, 128)
    sidx = jnp.concatenate(
        [receivers, jnp.full((pad,), N, jnp.int32)]).reshape(NS, TCHUNKS, 128)
    zeros_m = jnp.zeros((128, D), jnp.float32)

    nodes = x
    bf16 = jnp.bfloat16
    for s in range(2):
        W1 = W_e1[s]
        w1a = _pad8(W1[256:260].at[0].add(b_e1[s]))
        W2 = W_e2[s]
        w2a = _pad8(W2[128:132].at[0].add(b_e2[s]))
        w_edge = (W1[:128].astype(bf16), W1[128:256].astype(bf16),
                  W1[260:388].astype(bf16), W1[388:516].astype(bf16), w1a,
                  _pad8(Wy_e1[s][:, :128]), _pad8(Wy_e1[s][:, 128:]),
                  W2[:128].astype(bf16), W2[132:260].astype(bf16), w2a,
                  _pad8(Wy_e2[s]))
        w_node = (Wy_n1[s][:128].astype(bf16), _pad8(Wy_n1[s][128:132]),
                  W_n1[s][:128].astype(bf16), W_n1[s][128:256].astype(bf16),
                  W_n1[s][260:388].astype(bf16),
                  _pad8(W_n1[s][256:260]), b_n1[s][None, :],
                  Wy_n2[s][:128].astype(bf16), _pad8(Wy_n2[s][128:132]),
                  W_n2[s][:128].astype(bf16), W_n2[s][128:256].astype(bf16),
                  W_n2[s][260:388].astype(bf16),
                  _pad8(W_n2[s][256:260]), b_n2[s][None, :],
                  W_lin[s].astype(bf16), b_lin[s][None, :])

        sc_gather, sc_scatter = _sc_kernels()
        feat_tbl = jax.lax.bitcast_convert_type(
            nodes.astype(bf16).reshape(N, D // 2, 2), jnp.int32)
        pos_tbl = jnp.concatenate(
            [nodes[:, :3], jnp.zeros((N, PW - 3), jnp.float32)], axis=1)
        gfeat_i32, gpos = sc_gather(feat_tbl, pos_tbl, gidx)
        gfeat = jax.lax.bitcast_convert_type(
            gfeat_i32, bf16).reshape(2 * E_PAD, D)
        edata_m, edata_a = _edge_call(gfeat, gpos, w_edge)
        parts_m, parts_a = sc_scatter(edata_m, edata_a, sidx, zeros_m)
        nodes = _node_call(nodes, parts_m, parts_a, w_node)
    return nodes


# speedup vs baseline: 1.5630x; 1.5630x over previous
import functools

import jax
import jax.numpy as jnp
import numpy as np
from jax import lax
from jax.experimental import pallas as pl
from jax.experimental.pallas import tpu as pltpu
from jax.experimental.pallas import tpu_sc as plsc

N = 10000
E = 160000
D = 128

NC = 2
NS = 16
NW = NC * NS

E_PAD = 163840
GTOT = (2 * E_PAD) // 128
GC0 = 80
GC1 = GTOT // NS - GC0
PW = 16
TCHUNKS = E_PAD // (NS * 128)
ROWS_PER_TILE = 640
N_ACC = NS * ROWS_PER_TILE

EB = 512
NB = 400

_SR3 = float(np.sqrt(3.0))
_PERM = np.concatenate([np.arange(0, 128, 2), np.arange(1, 128, 2)])

@functools.lru_cache(maxsize=1)
def _sc_kernels():
    mesh = plsc.VectorSubcoreMesh(core_axis_name="c", subcore_axis_name="s")

    @functools.partial(
        pl.kernel,
        out_type=(jax.ShapeDtypeStruct((2 * E_PAD, D // 2), jnp.int32),
                  jax.ShapeDtypeStruct((2 * E_PAD, PW), jnp.float32)),
        mesh=mesh,
        compiler_params=pltpu.CompilerParams(use_tc_tiling_on_sc=False),
        scratch_types=[
            pltpu.VMEM((GC0, 128), jnp.int32),
            pltpu.VMEM((128, D // 2), jnp.int32),
            pltpu.VMEM((128, D // 2), jnp.int32),
            pltpu.VMEM((128, D // 2), jnp.int32),
            pltpu.VMEM((128, D // 2), jnp.int32),
            pltpu.VMEM((128, PW), jnp.float32),
            pltpu.VMEM((128, PW), jnp.float32),
            pltpu.VMEM((128, PW), jnp.float32),
            pltpu.VMEM((128, PW), jnp.float32),
            pltpu.SemaphoreType.DMA,
            pltpu.SemaphoreType.DMA,
            pltpu.SemaphoreType.DMA,
            pltpu.SemaphoreType.DMA,
            pltpu.SemaphoreType.DMA,
            pltpu.SemaphoreType.DMA,
            pltpu.SemaphoreType.DMA,
            pltpu.SemaphoreType.DMA,
        ],
    )
    def _sc_gather(feat_hbm, pos_hbm, idx_hbm, outf_hbm, outp_hbm, idx_v,
                   f0, f1, f2, f3, p0, p1, p2, p3,
                   s0, s1, s2, s3, w0, w1, w2, w3):
        fbufs = (f0, f1, f2, f3)
        pbufs = (p0, p1, p2, p3)
        sems = (s0, s1, s2, s3)
        wsems = (w0, w1, w2, w3)
        nbuf = 4
        cid = lax.axis_index("c")
        sid = lax.axis_index("s")

        def issue(j, b):
            pltpu.async_copy(feat_hbm.at[idx_v.at[j]], fbufs[b], sems[b])
            pltpu.async_copy(pos_hbm.at[idx_v.at[j]], pbufs[b], sems[b])

        def drain(b, semarr):
            pltpu.make_async_copy(
                feat_hbm.at[pl.ds(0, 128)], fbufs[b], semarr[b]).wait()
            pltpu.make_async_copy(
                pos_hbm.at[pl.ds(0, 128)], pbufs[b], semarr[b]).wait()

        def run(chunk0, nchunks):
            base = chunk0 * 128
            pltpu.sync_copy(idx_hbm.at[pl.ds(chunk0, nchunks)],
                            idx_v.at[pl.ds(0, nchunks)])
            for b in range(nbuf - 1):
                issue(b, b)

            def outer(g, _):
                for b in range(nbuf):
                    j = g * nbuf + b
                    pb = (b + 3) % nbuf
                    pj = j + 3

                    @pl.when(pj < nchunks)
                    def _():
                        @pl.when(j >= 1)
                        def _():
                            drain(pb, wsems)

                        issue(pj, pb)

                    drain(b, sems)
                    pltpu.async_copy(
                        fbufs[b], outf_hbm.at[pl.ds(base + j * 128, 128)],
                        wsems[b])
                    pltpu.async_copy(
                        pbufs[b], outp_hbm.at[pl.ds(base + j * 128, 128)],
                        wsems[b])

                return 0

            lax.fori_loop(0, nchunks // nbuf, outer, 0)
            for b in range(nbuf):
                drain(b, wsems)

        @pl.when(cid == 0)
        def _():
            run(sid * GC0, GC0)

        @pl.when(cid == 1)
        def _():
            run(NS * GC0 + sid * GC1, GC1)

    @functools.partial(
        pl.kernel,
        out_type=(jax.ShapeDtypeStruct((N_ACC, D), jnp.float32),
                  jax.ShapeDtypeStruct((N_ACC, D), jnp.float32)),
        mesh=mesh,
        scratch_types=[
            pltpu.VMEM((TCHUNKS, 128), jnp.int32),
            pltpu.VMEM((128, D), jnp.float32),
            pltpu.VMEM((128, D), jnp.float32),
            pltpu.VMEM_SHARED((N_ACC, D), jnp.float32),
            pltpu.SemaphoreType.DMA,
            pltpu.SemaphoreType.DMA,
        ],
    )
    def _sc_scatter(em_hbm, ea_hbm, sidx_hbm, z_hbm,
                    outm_hbm, outa_hbm, sidx_v, mbuf, mbuf2, acc, sem, sem2):
        cid = lax.axis_index("c")
        sid = lax.axis_index("s")
        row0 = sid * ROWS_PER_TILE

        pltpu.sync_copy(z_hbm, mbuf)

        def zbody(t, _):
            pltpu.sync_copy(mbuf, acc.at[pl.ds(row0 + t * 128, 128)])
            return 0

        lax.fori_loop(0, ROWS_PER_TILE // 128, zbody, 0)
        pltpu.sync_copy(sidx_hbm.at[sid], sidx_v)
        plsc.subcore_barrier()

        base = sid * (TCHUNKS * 128)

        def scatter_all(data_hbm):
            pltpu.async_copy(data_hbm.at[pl.ds(base, 128)], mbuf, sem)

            def body(g, _):
                for b in range(2):
                    j = 2 * g + b
                    buf, sm = (mbuf, sem) if b == 0 else (mbuf2, sem2)
                    nbuf, nsm = (mbuf2, sem2) if b == 0 else (mbuf, sem)
                    pltpu.make_async_copy(
                        data_hbm.at[pl.ds(0, 128)], buf, sm).wait()

                    @pl.when(j + 1 < TCHUNKS)
                    def _():
                        pltpu.async_copy(
                            data_hbm.at[pl.ds(base + (j + 1) * 128, 128)],
                            nbuf, nsm)

                    pltpu.sync_copy(buf, acc.at[sidx_v.at[j]], add=True)
                return 0

            lax.fori_loop(0, TCHUNKS // 2, body, 0)

        @pl.when(cid == 0)
        def _():
            scatter_all(em_hbm)

        @pl.when(cid == 1)
        def _():
            scatter_all(ea_hbm)

        plsc.subcore_barrier()

        def copy_out(out_hbm):
            def obody(t, _):
                pltpu.sync_copy(acc.at[pl.ds(row0 + t * 128, 128)], mbuf)
                pltpu.sync_copy(mbuf, out_hbm.at[pl.ds(row0 + t * 128, 128)])
                return 0
            lax.fori_loop(0, ROWS_PER_TILE // 128, obody, 0)

        @pl.when(cid == 0)
        def _():
            copy_out(outm_hbm)

        @pl.when(cid == 1)
        def _():
            copy_out(outa_hbm)

    return _sc_gather, _sc_scatter


def _edge_body(xs_ref, xr_ref, ps_ref, pr_ref, w1s, w1r, w1ps, w1pr, w1a,
               wy1s, wy1r, w2m, w2p, w2a, wy2, outm_ref, outa_ref):
    f32 = jnp.float32

    def unpack(xi):
        lo = jax.lax.bitcast_convert_type(xi << 16, f32)
        hi = jax.lax.bitcast_convert_type(xi & jnp.int32(-65536), f32)
        return jnp.concatenate([lo, hi], axis=-1)

    xs = unpack(xs_ref[...])
    xr = unpack(xr_ref[...])
    r = ps_ref[...][:, :3] - pr_ref[...][:, :3]
    d = jnp.sqrt(jnp.sum(r * r, axis=-1, keepdims=True))
    rh = (r / (d + 1e-8)) * _SR3
    a1, a2, a3 = rh[:, 0:1], rh[:, 1:2], rh[:, 2:3]

    def dot(x, w):
        return jax.lax.dot_general(x.astype(jnp.bfloat16), w[...],
                                   (((1,), (0,)), ((), ())),
                                   preferred_element_type=f32)

    y1s = wy1s[0:1] + a1 * wy1s[1:2] + a2 * wy1s[2:3] + a3 * wy1s[3:4]
    y1r = wy1r[0:1] + a1 * wy1r[1:2] + a2 * wy1r[2:3] + a3 * wy1r[3:4]
    pre = (dot(xs, w1s) + dot(xr, w1r) + dot(xs * y1s, w1ps) + dot(xr * y1r, w1pr)
           + w1a[0:1] + a1 * w1a[1:2] + a2 * w1a[2:3] + a3 * w1a[3:4])
    m = pre[:, 128:] * jax.nn.sigmoid(pre[:, :128])
    y2 = wy2[0:1] + a1 * wy2[1:2] + a2 * wy2[2:3] + a3 * wy2[3:4]
    pre2 = (dot(m, w2m) + dot(m * y2, w2p)
            + w2a[0:1] + a1 * w2a[1:2] + a2 * w2a[2:3] + a3 * w2a[3:4])
    m2 = pre2[:, 128:] * jax.nn.sigmoid(pre2[:, :128])
    ones = jnp.ones_like(a1)
    zeros = jnp.zeros((xs.shape[0], D - 4), f32)
    outm_ref[...] = m2
    outa_ref[...] = jnp.concatenate([a1, a2, a3, ones, zeros], axis=-1)


def _edge_call(gfeat, gpos, w):
    nblk = E_PAD // EB
    full = lambda arr: pl.BlockSpec(arr.shape, lambda i: (0,) * arr.ndim)
    return pl.pallas_call(
        _edge_body,
        grid=(nblk,),
        in_specs=[pl.BlockSpec((EB, D // 2), lambda i: (i, 0)),
                  pl.BlockSpec((EB, D // 2), lambda i: (i + nblk, 0)),
                  pl.BlockSpec((EB, PW), lambda i: (i, 0)),
                  pl.BlockSpec((EB, PW), lambda i: (i + nblk, 0))]
                 + [full(a) for a in w],
        out_specs=[pl.BlockSpec((EB, D), lambda i: (i, 0)),
                   pl.BlockSpec((EB, D), lambda i: (i, 0))],
        out_shape=[jax.ShapeDtypeStruct((E_PAD, D), jnp.float32),
                   jax.ShapeDtypeStruct((E_PAD, D), jnp.float32)],
    )(gfeat, gfeat, gpos, gpos, *w)


def _node_body(x_ref, pm_ref, pa_ref,
               wy1m, wy1a, wn1x, wn1m, wn1p, wn1a, b1,
               wy2m, wy2a, wn2x, wn2m, wn2p, wn2a, b2,
               wlin, blin, out_ref):
    f32 = jnp.float32
    x = x_ref[...]
    msum = pm_ref[...]
    asum = pa_ref[...]
    deg = asum[:, 3:4]
    inv = 1.0 / jnp.maximum(deg, 1.0)
    m_i = msum * inv
    ai0 = deg * inv
    ai1 = asum[:, 0:1] * inv
    ai2 = asum[:, 1:2] * inv
    ai3 = asum[:, 2:3] * inv

    def dot(a, w):
        return jax.lax.dot_general(a.astype(jnp.bfloat16), w[...],
                                   (((1,), (0,)), ((), ())),
                                   preferred_element_type=f32)

    def apart(wa):
        return (ai0 * wa[0:1] + ai1 * wa[1:2] + ai2 * wa[2:3] + ai3 * wa[3:4])

    y1 = dot(m_i, wy1m) + apart(wy1a)
    p1_ = (dot(x, wn1x) + dot(m_i, wn1m) + dot(x * y1, wn1p)
           + apart(wn1a) + b1[0:1])
    x1 = p1_[:, 128:] * jax.nn.sigmoid(p1_[:, :128])
    y2 = dot(m_i, wy2m) + apart(wy2a)
    p2_ = (dot(x1, wn2x) + dot(m_i, wn2m) + dot(x1 * y2, wn2p)
           + apart(wn2a) + b2[0:1])
    x2 = p2_[:, 128:] * jax.nn.sigmoid(p2_[:, :128])
    out_ref[...] = dot(x2, wlin) + blin[0:1]


def _node_call(nodes, parts_m, parts_a, w):
    nblk = N // NB
    full = lambda arr: pl.BlockSpec(arr.shape, lambda i: (0,) * arr.ndim)
    return pl.pallas_call(
        _node_body,
        grid=(nblk,),
        in_specs=[pl.BlockSpec((NB, D), lambda i: (i, 0)),
                  pl.BlockSpec((NB, D), lambda i: (i, 0)),
                  pl.BlockSpec((NB, D), lambda i: (i, 0))]
                 + [full(a) for a in w],
        out_specs=pl.BlockSpec((NB, D), lambda i: (i, 0)),
        out_shape=jax.ShapeDtypeStruct((N, D), jnp.float32),
    )(nodes, parts_m, parts_a, *w)


def _pad8(w):
    return jnp.concatenate([w, jnp.zeros((8 - w.shape[0],) + w.shape[1:], w.dtype)])


def kernel(x, edge_index, Wy_e1, W_e1, b_e1, Wy_e2, W_e2, b_e2,
           Wy_n1, W_n1, b_n1, Wy_n2, W_n2, b_n2, W_lin, b_lin):
    senders = edge_index[0].astype(jnp.int32)
    receivers = edge_index[1].astype(jnp.int32)
    pad = E_PAD - E
    spad = jnp.concatenate([senders, jnp.zeros((pad,), jnp.int32)])
    rpad = jnp.concatenate([receivers, jnp.zeros((pad,), jnp.int32)])
    gidx = jnp.concatenate([spad, rpad]).reshape(GTOT, 128)
    sidx = jnp.concatenate(
        [receivers, jnp.full((pad,), N, jnp.int32)]).reshape(NS, TCHUNKS, 128)
    zeros_m = jnp.zeros((128, D), jnp.float32)

    nodes = x
    bf16 = jnp.bfloat16
    for s in range(2):
        W1 = W_e1[s]
        w1a = _pad8(W1[256:260].at[0].add(b_e1[s]))
        W2 = W_e2[s]
        w2a = _pad8(W2[128:132].at[0].add(b_e2[s]))
        w_edge = (W1[:128][_PERM].astype(bf16), W1[128:256][_PERM].astype(bf16),
                  W1[260:388][_PERM].astype(bf16),
                  W1[388:516][_PERM].astype(bf16), w1a,
                  _pad8(Wy_e1[s][:, :128])[:, _PERM],
                  _pad8(Wy_e1[s][:, 128:])[:, _PERM],
                  W2[:128].astype(bf16), W2[132:260].astype(bf16), w2a,
                  _pad8(Wy_e2[s]))
        w_node = (Wy_n1[s][:128].astype(bf16), _pad8(Wy_n1[s][128:132]),
                  W_n1[s][:128].astype(bf16), W_n1[s][128:256].astype(bf16),
                  W_n1[s][260:388].astype(bf16),
                  _pad8(W_n1[s][256:260]), b_n1[s][None, :],
                  Wy_n2[s][:128].astype(bf16), _pad8(Wy_n2[s][128:132]),
                  W_n2[s][:128].astype(bf16), W_n2[s][128:256].astype(bf16),
                  W_n2[s][260:388].astype(bf16),
                  _pad8(W_n2[s][256:260]), b_n2[s][None, :],
                  W_lin[s].astype(bf16), b_lin[s][None, :])

        sc_gather, sc_scatter = _sc_kernels()
        feat_tbl = jax.lax.bitcast_convert_type(
            nodes.astype(bf16).reshape(N, D // 2, 2), jnp.int32)
        pos_tbl = jnp.concatenate(
            [nodes[:, :3], jnp.zeros((N, PW - 3), jnp.float32)], axis=1)
        gfeat_i32, gpos = sc_gather(feat_tbl, pos_tbl, gidx)
        edata_m, edata_a = _edge_call(gfeat_i32, gpos, w_edge)
        parts_m, parts_a = sc_scatter(edata_m, edata_a, sidx, zeros_m)
        nodes = _node_call(nodes, parts_m, parts_a, w_node)
    return nodes


# speedup vs baseline: 1.7851x; 1.1421x over previous
import functools

import jax
import jax.numpy as jnp
import numpy as np
from jax import lax
from jax.experimental import pallas as pl
from jax.experimental.pallas import tpu as pltpu
from jax.experimental.pallas import tpu_sc as plsc

N = 10000
E = 160000
D = 128

NC = 2
NS = 16
NW = NC * NS

E_PAD = 163840
GTOT = (2 * E_PAD) // 128
GC0 = 80
GC1 = GTOT // NS - GC0
PW = 16
TCHUNKS = E_PAD // (NS * 128)
ROWS_PER_TILE = 640
N_ACC = NS * ROWS_PER_TILE

EB = 512
NB = 400

_SR3 = float(np.sqrt(3.0))
_PERM = np.concatenate([np.arange(0, 128, 2), np.arange(1, 128, 2)])

@functools.lru_cache(maxsize=1)
def _sc_kernels():
    mesh = plsc.VectorSubcoreMesh(core_axis_name="c", subcore_axis_name="s")

    @functools.partial(
        pl.kernel,
        out_type=jax.ShapeDtypeStruct((2 * E_PAD, D), jnp.float32),
        mesh=mesh,
        scratch_types=[
            pltpu.VMEM((GC0, 128), jnp.int32),
            pltpu.VMEM((128, D), jnp.float32),
            pltpu.VMEM((128, D), jnp.float32),
            pltpu.VMEM((128, D), jnp.float32),
            pltpu.VMEM((128, D), jnp.float32),
            pltpu.SemaphoreType.DMA,
            pltpu.SemaphoreType.DMA,
            pltpu.SemaphoreType.DMA,
            pltpu.SemaphoreType.DMA,
            pltpu.SemaphoreType.DMA,
            pltpu.SemaphoreType.DMA,
            pltpu.SemaphoreType.DMA,
            pltpu.SemaphoreType.DMA,
        ],
    )
    def _sc_gather(nodes_hbm, idx_hbm, out_hbm, idx_v,
                   b0, b1, b2, b3, s0, s1, s2, s3, w0, w1, w2, w3):
        bufs = (b0, b1, b2, b3)
        sems = (s0, s1, s2, s3)
        wsems = (w0, w1, w2, w3)
        nbuf = 4
        cid = lax.axis_index("c")
        sid = lax.axis_index("s")

        def run(chunk0, nchunks):
            base = chunk0 * 128
            pltpu.sync_copy(idx_hbm.at[pl.ds(chunk0, nchunks)],
                            idx_v.at[pl.ds(0, nchunks)])
            for b in range(nbuf - 1):
                pltpu.async_copy(nodes_hbm.at[idx_v.at[b]], bufs[b], sems[b])

            def outer(g, _):
                for b in range(nbuf):
                    j = g * nbuf + b
                    pb = (b + 3) % nbuf
                    pj = j + 3

                    @pl.when(pj < nchunks)
                    def _():
                        @pl.when(j >= 1)
                        def _():
                            pltpu.make_async_copy(
                                nodes_hbm.at[pl.ds(0, 128)],
                                bufs[pb], wsems[pb]).wait()

                        pltpu.async_copy(
                            nodes_hbm.at[idx_v.at[pj]], bufs[pb], sems[pb])

                    pltpu.make_async_copy(
                        nodes_hbm.at[pl.ds(0, 128)], bufs[b], sems[b]).wait()
                    pltpu.async_copy(bufs[b],
                                     out_hbm.at[pl.ds(base + j * 128, 128)],
                                     wsems[b])

                return 0

            lax.fori_loop(0, nchunks // nbuf, outer, 0)
            for b in range(nbuf):
                pltpu.make_async_copy(
                    nodes_hbm.at[pl.ds(0, 128)], bufs[b], wsems[b]).wait()

        @pl.when(cid == 0)
        def _():
            run(sid * GC0, GC0)

        @pl.when(cid == 1)
        def _():
            run(NS * GC0 + sid * GC1, GC1)

    @functools.partial(
        pl.kernel,
        out_type=(jax.ShapeDtypeStruct((N_ACC, D), jnp.float32),
                  jax.ShapeDtypeStruct((N_ACC, D), jnp.float32)),
        mesh=mesh,
        scratch_types=[
            pltpu.VMEM((TCHUNKS, 128), jnp.int32),
            pltpu.VMEM((128, D), jnp.float32),
            pltpu.VMEM((128, D), jnp.float32),
            pltpu.VMEM_SHARED((N_ACC, D), jnp.float32),
            pltpu.SemaphoreType.DMA,
            pltpu.SemaphoreType.DMA,
        ],
    )
    def _sc_scatter(em_hbm, ea_hbm, sidx_hbm, z_hbm,
                    outm_hbm, outa_hbm, sidx_v, mbuf, mbuf2, acc, sem, sem2):
        cid = lax.axis_index("c")
        sid = lax.axis_index("s")
        row0 = sid * ROWS_PER_TILE

        pltpu.sync_copy(z_hbm, mbuf)

        def zbody(t, _):
            pltpu.sync_copy(mbuf, acc.at[pl.ds(row0 + t * 128, 128)])
            return 0

        lax.fori_loop(0, ROWS_PER_TILE // 128, zbody, 0)
        pltpu.sync_copy(sidx_hbm.at[sid], sidx_v)
        plsc.subcore_barrier()

        base = sid * (TCHUNKS * 128)

        def scatter_all(data_hbm):
            pltpu.async_copy(data_hbm.at[pl.ds(base, 128)], mbuf, sem)

            def body(g, _):
                for b in range(2):
                    j = 2 * g + b
                    buf, sm = (mbuf, sem) if b == 0 else (mbuf2, sem2)
                    nbuf, nsm = (mbuf2, sem2) if b == 0 else (mbuf, sem)
                    pltpu.make_async_copy(
                        data_hbm.at[pl.ds(0, 128)], buf, sm).wait()

                    @pl.when(j + 1 < TCHUNKS)
                    def _():
                        pltpu.async_copy(
                            data_hbm.at[pl.ds(base + (j + 1) * 128, 128)],
                            nbuf, nsm)

                    pltpu.sync_copy(buf, acc.at[sidx_v.at[j]], add=True)
                return 0

            lax.fori_loop(0, TCHUNKS // 2, body, 0)

        @pl.when(cid == 0)
        def _():
            scatter_all(em_hbm)

        @pl.when(cid == 1)
        def _():
            scatter_all(ea_hbm)

        plsc.subcore_barrier()

        def copy_out(out_hbm):
            def obody(t, _):
                pltpu.sync_copy(acc.at[pl.ds(row0 + t * 128, 128)], mbuf)
                pltpu.sync_copy(mbuf, out_hbm.at[pl.ds(row0 + t * 128, 128)])
                return 0
            lax.fori_loop(0, ROWS_PER_TILE // 128, obody, 0)

        @pl.when(cid == 0)
        def _():
            copy_out(outm_hbm)

        @pl.when(cid == 1)
        def _():
            copy_out(outa_hbm)

    return _sc_gather, _sc_scatter


def _edge_body(xs_ref, xr_ref, w1s, w1r, w1ps, w1pr, w1a,
               wy1s, wy1r, w2m, w2p, w2a, wy2, outm_ref, outa_ref):
    f32 = jnp.float32
    xs = xs_ref[...]
    xr = xr_ref[...]
    r = xs[:, :3] - xr[:, :3]
    d = jnp.sqrt(jnp.sum(r * r, axis=-1, keepdims=True))
    rh = (r / (d + 1e-8)) * _SR3
    a1, a2, a3 = rh[:, 0:1], rh[:, 1:2], rh[:, 2:3]

    def dot(x, w):
        return jax.lax.dot_general(x.astype(jnp.bfloat16), w[...],
                                   (((1,), (0,)), ((), ())),
                                   preferred_element_type=f32)

    y1s = wy1s[0:1] + a1 * wy1s[1:2] + a2 * wy1s[2:3] + a3 * wy1s[3:4]
    y1r = wy1r[0:1] + a1 * wy1r[1:2] + a2 * wy1r[2:3] + a3 * wy1r[3:4]
    pre = (dot(xs, w1s) + dot(xr, w1r) + dot(xs * y1s, w1ps) + dot(xr * y1r, w1pr)
           + w1a[0:1] + a1 * w1a[1:2] + a2 * w1a[2:3] + a3 * w1a[3:4])
    m = pre[:, 128:] * jax.nn.sigmoid(pre[:, :128])
    y2 = wy2[0:1] + a1 * wy2[1:2] + a2 * wy2[2:3] + a3 * wy2[3:4]
    pre2 = (dot(m, w2m) + dot(m * y2, w2p)
            + w2a[0:1] + a1 * w2a[1:2] + a2 * w2a[2:3] + a3 * w2a[3:4])
    m2 = pre2[:, 128:] * jax.nn.sigmoid(pre2[:, :128])
    ones = jnp.ones_like(a1)
    zeros = jnp.zeros((xs.shape[0], D - 4), f32)
    outm_ref[...] = m2
    outa_ref[...] = jnp.concatenate([a1, a2, a3, ones, zeros], axis=-1)


def _edge_call(gathered, w):
    nblk = E_PAD // EB
    full = lambda arr: pl.BlockSpec(arr.shape, lambda i: (0,) * arr.ndim)
    return pl.pallas_call(
        _edge_body,
        grid=(nblk,),
        in_specs=[pl.BlockSpec((EB, D), lambda i: (i, 0)),
                  pl.BlockSpec((EB, D), lambda i: (i + nblk, 0))]
                 + [full(a) for a in w],
        out_specs=[pl.BlockSpec((EB, D), lambda i: (i, 0)),
                   pl.BlockSpec((EB, D), lambda i: (i, 0))],
        out_shape=[jax.ShapeDtypeStruct((E_PAD, D), jnp.float32),
                   jax.ShapeDtypeStruct((E_PAD, D), jnp.float32)],
    )(gathered, gathered, *w)


def _node_body(x_ref, pm_ref, pa_ref,
               wy1m, wy1a, wn1x, wn1m, wn1p, wn1a, b1,
               wy2m, wy2a, wn2x, wn2m, wn2p, wn2a, b2,
               wlin, blin, out_ref):
    f32 = jnp.float32
    x = x_ref[...]
    msum = pm_ref[...]
    asum = pa_ref[...]
    deg = asum[:, 3:4]
    inv = 1.0 / jnp.maximum(deg, 1.0)
    m_i = msum * inv
    ai0 = deg * inv
    ai1 = asum[:, 0:1] * inv
    ai2 = asum[:, 1:2] * inv
    ai3 = asum[:, 2:3] * inv

    def dot(a, w):
        return jax.lax.dot_general(a.astype(jnp.bfloat16), w[...],
                                   (((1,), (0,)), ((), ())),
                                   preferred_element_type=f32)

    def apart(wa):
        return (ai0 * wa[0:1] + ai1 * wa[1:2] + ai2 * wa[2:3] + ai3 * wa[3:4])

    y1 = dot(m_i, wy1m) + apart(wy1a)
    p1_ = (dot(x, wn1x) + dot(m_i, wn1m) + dot(x * y1, wn1p)
           + apart(wn1a) + b1[0:1])
    x1 = p1_[:, 128:] * jax.nn.sigmoid(p1_[:, :128])
    y2 = dot(m_i, wy2m) + apart(wy2a)
    p2_ = (dot(x1, wn2x) + dot(m_i, wn2m) + dot(x1 * y2, wn2p)
           + apart(wn2a) + b2[0:1])
    x2 = p2_[:, 128:] * jax.nn.sigmoid(p2_[:, :128])
    out_ref[...] = dot(x2, wlin) + blin[0:1]


def _node_call(nodes, parts_m, parts_a, w):
    nblk = N // NB
    full = lambda arr: pl.BlockSpec(arr.shape, lambda i: (0,) * arr.ndim)
    return pl.pallas_call(
        _node_body,
        grid=(nblk,),
        in_specs=[pl.BlockSpec((NB, D), lambda i: (i, 0)),
                  pl.BlockSpec((NB, D), lambda i: (i, 0)),
                  pl.BlockSpec((NB, D), lambda i: (i, 0))]
                 + [full(a) for a in w],
        out_specs=pl.BlockSpec((NB, D), lambda i: (i, 0)),
        out_shape=jax.ShapeDtypeStruct((N, D), jnp.float32),
    )(nodes, parts_m, parts_a, *w)


def _pad8(w):
    return jnp.concatenate([w, jnp.zeros((8 - w.shape[0],) + w.shape[1:], w.dtype)])


def kernel(x, edge_index, Wy_e1, W_e1, b_e1, Wy_e2, W_e2, b_e2,
           Wy_n1, W_n1, b_n1, Wy_n2, W_n2, b_n2, W_lin, b_lin):
    senders = edge_index[0].astype(jnp.int32)
    receivers = edge_index[1].astype(jnp.int32)
    pad = E_PAD - E
    spad = jnp.concatenate([senders, jnp.zeros((pad,), jnp.int32)])
    rpad = jnp.concatenate([receivers, jnp.zeros((pad,), jnp.int32)])
    gidx = jnp.concatenate([spad, rpad]).reshape(GTOT, 128)
    sidx = jnp.concatenate(
        [receivers, jnp.full((pad,), N, jnp.int32)]).reshape(NS, TCHUNKS, 128)
    zeros_m = jnp.zeros((128, D), jnp.float32)

    nodes = x
    bf16 = jnp.bfloat16
    for s in range(2):
        W1 = W_e1[s]
        w1a = _pad8(W1[256:260].at[0].add(b_e1[s]))
        W2 = W_e2[s]
        w2a = _pad8(W2[128:132].at[0].add(b_e2[s]))
        w_edge = (W1[:128].astype(bf16), W1[128:256].astype(bf16),
                  W1[260:388].astype(bf16), W1[388:516].astype(bf16), w1a,
                  _pad8(Wy_e1[s][:, :128]), _pad8(Wy_e1[s][:, 128:]),
                  W2[:128].astype(bf16), W2[132:260].astype(bf16), w2a,
                  _pad8(Wy_e2[s]))
        w_node = (Wy_n1[s][:128].astype(bf16), _pad8(Wy_n1[s][128:132]),
                  W_n1[s][:128].astype(bf16), W_n1[s][128:256].astype(bf16),
                  W_n1[s][260:388].astype(bf16),
                  _pad8(W_n1[s][256:260]), b_n1[s][None, :],
                  Wy_n2[s][:128].astype(bf16), _pad8(Wy_n2[s][128:132]),
                  W_n2[s][:128].astype(bf16), W_n2[s][128:256].astype(bf16),
                  W_n2[s][260:388].astype(bf16),
                  _pad8(W_n2[s][256:260]), b_n2[s][None, :],
                  W_lin[s].astype(bf16), b_lin[s][None, :])

        sc_gather, sc_scatter = _sc_kernels()
        gathered = sc_gather(nodes, gidx)
        edata_m, edata_a = _edge_call(gathered, w_edge)
        parts_m, parts_a = sc_scatter(edata_m, edata_a, sidx, zeros_m)
        nodes = _node_call(nodes, parts_m, parts_a, w_node)
    return nodes


# speedup vs baseline: 1.8588x; 1.0412x over previous
import functools

import jax
import jax.numpy as jnp
import numpy as np
from jax import lax
from jax.experimental import pallas as pl
from jax.experimental.pallas import tpu as pltpu
from jax.experimental.pallas import tpu_sc as plsc

N = 10000
E = 160000
D = 128

NC = 2
NS = 16
NW = NC * NS

E_PAD = 163840
GTOT = (2 * E_PAD) // 128
GC0 = 80
GC1 = GTOT // NS - GC0
PW = 16
TCHUNKS = E_PAD // (NS * 128)
ROWS_PER_TILE = 640
N_ACC = NS * ROWS_PER_TILE

EB = 1024
NB = 400

_SR3 = float(np.sqrt(3.0))
_PERM = np.concatenate([np.arange(0, 128, 2), np.arange(1, 128, 2)])

@functools.lru_cache(maxsize=1)
def _sc_kernels():
    mesh = plsc.VectorSubcoreMesh(core_axis_name="c", subcore_axis_name="s")

    @functools.partial(
        pl.kernel,
        out_type=jax.ShapeDtypeStruct((2 * E_PAD, D), jnp.float32),
        mesh=mesh,
        scratch_types=[
            pltpu.VMEM((GC0, 128), jnp.int32),
            pltpu.VMEM((128, D), jnp.float32),
            pltpu.VMEM((128, D), jnp.float32),
            pltpu.VMEM((128, D), jnp.float32),
            pltpu.VMEM((128, D), jnp.float32),
            pltpu.SemaphoreType.DMA,
            pltpu.SemaphoreType.DMA,
            pltpu.SemaphoreType.DMA,
            pltpu.SemaphoreType.DMA,
            pltpu.SemaphoreType.DMA,
            pltpu.SemaphoreType.DMA,
            pltpu.SemaphoreType.DMA,
            pltpu.SemaphoreType.DMA,
        ],
    )
    def _sc_gather(nodes_hbm, idx_hbm, out_hbm, idx_v,
                   b0, b1, b2, b3, s0, s1, s2, s3, w0, w1, w2, w3):
        bufs = (b0, b1, b2, b3)
        sems = (s0, s1, s2, s3)
        wsems = (w0, w1, w2, w3)
        nbuf = 4
        cid = lax.axis_index("c")
        sid = lax.axis_index("s")

        def run(chunk0, nchunks):
            base = chunk0 * 128
            pltpu.sync_copy(idx_hbm.at[pl.ds(chunk0, nchunks)],
                            idx_v.at[pl.ds(0, nchunks)])
            for b in range(nbuf - 1):
                pltpu.async_copy(nodes_hbm.at[idx_v.at[b]], bufs[b], sems[b])

            def outer(g, _):
                for b in range(nbuf):
                    j = g * nbuf + b
                    pb = (b + 3) % nbuf
                    pj = j + 3

                    @pl.when(pj < nchunks)
                    def _():
                        @pl.when(j >= 1)
                        def _():
                            pltpu.make_async_copy(
                                nodes_hbm.at[pl.ds(0, 128)],
                                bufs[pb], wsems[pb]).wait()

                        pltpu.async_copy(
                            nodes_hbm.at[idx_v.at[pj]], bufs[pb], sems[pb])

                    pltpu.make_async_copy(
                        nodes_hbm.at[pl.ds(0, 128)], bufs[b], sems[b]).wait()
                    pltpu.async_copy(bufs[b],
                                     out_hbm.at[pl.ds(base + j * 128, 128)],
                                     wsems[b])

                return 0

            lax.fori_loop(0, nchunks // nbuf, outer, 0)
            for b in range(nbuf):
                pltpu.make_async_copy(
                    nodes_hbm.at[pl.ds(0, 128)], bufs[b], wsems[b]).wait()

        @pl.when(cid == 0)
        def _():
            run(sid * GC0, GC0)

        @pl.when(cid == 1)
        def _():
            run(NS * GC0 + sid * GC1, GC1)

    @functools.partial(
        pl.kernel,
        out_type=(jax.ShapeDtypeStruct((N_ACC, D), jnp.float32),
                  jax.ShapeDtypeStruct((N_ACC, D), jnp.float32)),
        mesh=mesh,
        scratch_types=[
            pltpu.VMEM((TCHUNKS, 128), jnp.int32),
            pltpu.VMEM((128, D), jnp.float32),
            pltpu.VMEM((128, D), jnp.float32),
            pltpu.VMEM_SHARED((N_ACC, D), jnp.float32),
            pltpu.SemaphoreType.DMA,
            pltpu.SemaphoreType.DMA,
        ],
    )
    def _sc_scatter(em_hbm, ea_hbm, sidx_hbm, z_hbm,
                    outm_hbm, outa_hbm, sidx_v, mbuf, mbuf2, acc, sem, sem2):
        cid = lax.axis_index("c")
        sid = lax.axis_index("s")
        row0 = sid * ROWS_PER_TILE

        pltpu.sync_copy(z_hbm, mbuf)

        def zbody(t, _):
            pltpu.sync_copy(mbuf, acc.at[pl.ds(row0 + t * 128, 128)])
            return 0

        lax.fori_loop(0, ROWS_PER_TILE // 128, zbody, 0)
        pltpu.sync_copy(sidx_hbm.at[sid], sidx_v)
        plsc.subcore_barrier()

        base = sid * (TCHUNKS * 128)

        def scatter_all(data_hbm):
            pltpu.async_copy(data_hbm.at[pl.ds(base, 128)], mbuf, sem)

            def body(g, _):
                for b in range(2):
                    j = 2 * g + b
                    buf, sm = (mbuf, sem) if b == 0 else (mbuf2, sem2)
                    nbuf, nsm = (mbuf2, sem2) if b == 0 else (mbuf, sem)
                    pltpu.make_async_copy(
                        data_hbm.at[pl.ds(0, 128)], buf, sm).wait()

                    @pl.when(j + 1 < TCHUNKS)
                    def _():
                        pltpu.async_copy(
                            data_hbm.at[pl.ds(base + (j + 1) * 128, 128)],
                            nbuf, nsm)

                    pltpu.sync_copy(buf, acc.at[sidx_v.at[j]], add=True)
                return 0

            lax.fori_loop(0, TCHUNKS // 2, body, 0)

        @pl.when(cid == 0)
        def _():
            scatter_all(em_hbm)

        @pl.when(cid == 1)
        def _():
            scatter_all(ea_hbm)

        plsc.subcore_barrier()

        def copy_out(out_hbm):
            def obody(t, _):
                pltpu.sync_copy(acc.at[pl.ds(row0 + t * 128, 128)], mbuf)
                pltpu.sync_copy(mbuf, out_hbm.at[pl.ds(row0 + t * 128, 128)])
                return 0
            lax.fori_loop(0, ROWS_PER_TILE // 128, obody, 0)

        @pl.when(cid == 0)
        def _():
            copy_out(outm_hbm)

        @pl.when(cid == 1)
        def _():
            copy_out(outa_hbm)

    return _sc_gather, _sc_scatter


def _edge_body(xs_ref, xr_ref, w1s, w1r, w1ps, w1pr, w1a,
               wy1s, wy1r, w2m, w2p, w2a, wy2, outm_ref, outa_ref):
    f32 = jnp.float32
    xs = xs_ref[...]
    xr = xr_ref[...]
    r = xs[:, :3] - xr[:, :3]
    d = jnp.sqrt(jnp.sum(r * r, axis=-1, keepdims=True))
    rh = (r / (d + 1e-8)) * _SR3
    a1, a2, a3 = rh[:, 0:1], rh[:, 1:2], rh[:, 2:3]

    def dot(x, w):
        return jax.lax.dot_general(x.astype(jnp.bfloat16), w[...],
                                   (((1,), (0,)), ((), ())),
                                   preferred_element_type=f32)

    y1s = wy1s[0:1] + a1 * wy1s[1:2] + a2 * wy1s[2:3] + a3 * wy1s[3:4]
    y1r = wy1r[0:1] + a1 * wy1r[1:2] + a2 * wy1r[2:3] + a3 * wy1r[3:4]
    pre = (dot(xs, w1s) + dot(xr, w1r) + dot(xs * y1s, w1ps) + dot(xr * y1r, w1pr)
           + w1a[0:1] + a1 * w1a[1:2] + a2 * w1a[2:3] + a3 * w1a[3:4])
    m = pre[:, 128:] * jax.nn.sigmoid(pre[:, :128])
    y2 = wy2[0:1] + a1 * wy2[1:2] + a2 * wy2[2:3] + a3 * wy2[3:4]
    pre2 = (dot(m, w2m) + dot(m * y2, w2p)
            + w2a[0:1] + a1 * w2a[1:2] + a2 * w2a[2:3] + a3 * w2a[3:4])
    m2 = pre2[:, 128:] * jax.nn.sigmoid(pre2[:, :128])
    ones = jnp.ones_like(a1)
    zeros = jnp.zeros((xs.shape[0], D - 4), f32)
    outm_ref[...] = m2
    outa_ref[...] = jnp.concatenate([a1, a2, a3, ones, zeros], axis=-1)


def _edge_call(gathered, w):
    nblk = E_PAD // EB
    full = lambda arr: pl.BlockSpec(arr.shape, lambda i: (0,) * arr.ndim)
    return pl.pallas_call(
        _edge_body,
        grid=(nblk,),
        in_specs=[pl.BlockSpec((EB, D), lambda i: (i, 0)),
                  pl.BlockSpec((EB, D), lambda i: (i + nblk, 0))]
                 + [full(a) for a in w],
        out_specs=[pl.BlockSpec((EB, D), lambda i: (i, 0)),
                   pl.BlockSpec((EB, D), lambda i: (i, 0))],
        out_shape=[jax.ShapeDtypeStruct((E_PAD, D), jnp.float32),
                   jax.ShapeDtypeStruct((E_PAD, D), jnp.float32)],
    )(gathered, gathered, *w)


def _node_body(x_ref, pm_ref, pa_ref,
               wy1m, wy1a, wn1x, wn1m, wn1p, wn1a, b1,
               wy2m, wy2a, wn2x, wn2m, wn2p, wn2a, b2,
               wlin, blin, out_ref):
    f32 = jnp.float32
    x = x_ref[...]
    msum = pm_ref[...]
    asum = pa_ref[...]
    deg = asum[:, 3:4]
    inv = 1.0 / jnp.maximum(deg, 1.0)
    m_i = msum * inv
    ai0 = deg * inv
    ai1 = asum[:, 0:1] * inv
    ai2 = asum[:, 1:2] * inv
    ai3 = asum[:, 2:3] * inv

    def dot(a, w):
        return jax.lax.dot_general(a.astype(jnp.bfloat16), w[...],
                                   (((1,), (0,)), ((), ())),
                                   preferred_element_type=f32)

    def apart(wa):
        return (ai0 * wa[0:1] + ai1 * wa[1:2] + ai2 * wa[2:3] + ai3 * wa[3:4])

    y1 = dot(m_i, wy1m) + apart(wy1a)
    p1_ = (dot(x, wn1x) + dot(m_i, wn1m) + dot(x * y1, wn1p)
           + apart(wn1a) + b1[0:1])
    x1 = p1_[:, 128:] * jax.nn.sigmoid(p1_[:, :128])
    y2 = dot(m_i, wy2m) + apart(wy2a)
    p2_ = (dot(x1, wn2x) + dot(m_i, wn2m) + dot(x1 * y2, wn2p)
           + apart(wn2a) + b2[0:1])
    x2 = p2_[:, 128:] * jax.nn.sigmoid(p2_[:, :128])
    out_ref[...] = dot(x2, wlin) + blin[0:1]


def _node_call(nodes, parts_m, parts_a, w):
    nblk = N // NB
    full = lambda arr: pl.BlockSpec(arr.shape, lambda i: (0,) * arr.ndim)
    return pl.pallas_call(
        _node_body,
        grid=(nblk,),
        in_specs=[pl.BlockSpec((NB, D), lambda i: (i, 0)),
                  pl.BlockSpec((NB, D), lambda i: (i, 0)),
                  pl.BlockSpec((NB, D), lambda i: (i, 0))]
                 + [full(a) for a in w],
        out_specs=pl.BlockSpec((NB, D), lambda i: (i, 0)),
        out_shape=jax.ShapeDtypeStruct((N, D), jnp.float32),
    )(nodes, parts_m, parts_a, *w)


def _pad8(w):
    return jnp.concatenate([w, jnp.zeros((8 - w.shape[0],) + w.shape[1:], w.dtype)])


def kernel(x, edge_index, Wy_e1, W_e1, b_e1, Wy_e2, W_e2, b_e2,
           Wy_n1, W_n1, b_n1, Wy_n2, W_n2, b_n2, W_lin, b_lin):
    senders = edge_index[0].astype(jnp.int32)
    receivers = edge_index[1].astype(jnp.int32)
    pad = E_PAD - E
    spad = jnp.concatenate([senders, jnp.zeros((pad,), jnp.int32)])
    rpad = jnp.concatenate([receivers, jnp.zeros((pad,), jnp.int32)])
    gidx = jnp.concatenate([spad, rpad]).reshape(GTOT, 128)
    sidx = jnp.concatenate(
        [receivers, jnp.full((pad,), N, jnp.int32)]).reshape(NS, TCHUNKS, 128)
    zeros_m = jnp.zeros((128, D), jnp.float32)

    nodes = x
    bf16 = jnp.bfloat16
    for s in range(2):
        W1 = W_e1[s]
        w1a = _pad8(W1[256:260].at[0].add(b_e1[s]))
        W2 = W_e2[s]
        w2a = _pad8(W2[128:132].at[0].add(b_e2[s]))
        w_edge = (W1[:128].astype(bf16), W1[128:256].astype(bf16),
                  W1[260:388].astype(bf16), W1[388:516].astype(bf16), w1a,
                  _pad8(Wy_e1[s][:, :128]), _pad8(Wy_e1[s][:, 128:]),
                  W2[:128].astype(bf16), W2[132:260].astype(bf16), w2a,
                  _pad8(Wy_e2[s]))
        w_node = (Wy_n1[s][:128].astype(bf16), _pad8(Wy_n1[s][128:132]),
                  W_n1[s][:128].astype(bf16), W_n1[s][128:256].astype(bf16),
                  W_n1[s][260:388].astype(bf16),
                  _pad8(W_n1[s][256:260]), b_n1[s][None, :],
                  Wy_n2[s][:128].astype(bf16), _pad8(Wy_n2[s][128:132]),
                  W_n2[s][:128].astype(bf16), W_n2[s][128:256].astype(bf16),
                  W_n2[s][260:388].astype(bf16),
                  _pad8(W_n2[s][256:260]), b_n2[s][None, :],
                  W_lin[s].astype(bf16), b_lin[s][None, :])

        sc_gather, sc_scatter = _sc_kernels()
        gathered = sc_gather(nodes, gidx)
        edata_m, edata_a = _edge_call(gathered, w_edge)
        parts_m, parts_a = sc_scatter(edata_m, edata_a, sidx, zeros_m)
        nodes = _node_call(nodes, parts_m, parts_a, w_node)
    return nodes


# speedup vs baseline: 1.9847x; 1.0677x over previous
import functools

import jax
import jax.numpy as jnp
import numpy as np
from jax import lax
from jax.experimental import pallas as pl
from jax.experimental.pallas import tpu as pltpu
from jax.experimental.pallas import tpu_sc as plsc

N = 10000
E = 160000
D = 128

NC = 2
NS = 16
NW = NC * NS

E_PAD = 163840
GTOT = (2 * E_PAD) // 128
GC0 = 80
GC1 = GTOT // NS - GC0
PW = 16
TCHUNKS = E_PAD // (NS * 128)
ROWS_PER_TILE = 640
N_ACC = NS * ROWS_PER_TILE

EB = 2048
NB = 1000

_SR3 = float(np.sqrt(3.0))
_PERM = np.concatenate([np.arange(0, 128, 2), np.arange(1, 128, 2)])

@functools.lru_cache(maxsize=1)
def _sc_kernels():
    mesh = plsc.VectorSubcoreMesh(core_axis_name="c", subcore_axis_name="s")

    @functools.partial(
        pl.kernel,
        out_type=jax.ShapeDtypeStruct((2 * E_PAD, D), jnp.float32),
        mesh=mesh,
        scratch_types=[
            pltpu.VMEM((GC0, 128), jnp.int32),
            pltpu.VMEM((128, D), jnp.float32),
            pltpu.VMEM((128, D), jnp.float32),
            pltpu.VMEM((128, D), jnp.float32),
            pltpu.VMEM((128, D), jnp.float32),
            pltpu.SemaphoreType.DMA,
            pltpu.SemaphoreType.DMA,
            pltpu.SemaphoreType.DMA,
            pltpu.SemaphoreType.DMA,
            pltpu.SemaphoreType.DMA,
            pltpu.SemaphoreType.DMA,
            pltpu.SemaphoreType.DMA,
            pltpu.SemaphoreType.DMA,
        ],
    )
    def _sc_gather(nodes_hbm, idx_hbm, out_hbm, idx_v,
                   b0, b1, b2, b3, s0, s1, s2, s3, w0, w1, w2, w3):
        bufs = (b0, b1, b2, b3)
        sems = (s0, s1, s2, s3)
        wsems = (w0, w1, w2, w3)
        nbuf = 4
        cid = lax.axis_index("c")
        sid = lax.axis_index("s")

        def run(chunk0, nchunks):
            base = chunk0 * 128
            pltpu.sync_copy(idx_hbm.at[pl.ds(chunk0, nchunks)],
                            idx_v.at[pl.ds(0, nchunks)])
            for b in range(nbuf - 1):
                pltpu.async_copy(nodes_hbm.at[idx_v.at[b]], bufs[b], sems[b])

            def outer(g, _):
                for b in range(nbuf):
                    j = g * nbuf + b
                    pb = (b + 3) % nbuf
                    pj = j + 3

                    @pl.when(pj < nchunks)
                    def _():
                        @pl.when(j >= 1)
                        def _():
                            pltpu.make_async_copy(
                                nodes_hbm.at[pl.ds(0, 128)],
                                bufs[pb], wsems[pb]).wait()

                        pltpu.async_copy(
                            nodes_hbm.at[idx_v.at[pj]], bufs[pb], sems[pb])

                    pltpu.make_async_copy(
                        nodes_hbm.at[pl.ds(0, 128)], bufs[b], sems[b]).wait()
                    pltpu.async_copy(bufs[b],
                                     out_hbm.at[pl.ds(base + j * 128, 128)],
                                     wsems[b])

                return 0

            lax.fori_loop(0, nchunks // nbuf, outer, 0)
            for b in range(nbuf):
                pltpu.make_async_copy(
                    nodes_hbm.at[pl.ds(0, 128)], bufs[b], wsems[b]).wait()

        @pl.when(cid == 0)
        def _():
            run(sid * GC0, GC0)

        @pl.when(cid == 1)
        def _():
            run(NS * GC0 + sid * GC1, GC1)

    @functools.partial(
        pl.kernel,
        out_type=(jax.ShapeDtypeStruct((N_ACC, D), jnp.float32),
                  jax.ShapeDtypeStruct((N_ACC, D), jnp.float32)),
        mesh=mesh,
        scratch_types=[
            pltpu.VMEM((TCHUNKS, 128), jnp.int32),
            pltpu.VMEM((128, D), jnp.float32),
            pltpu.VMEM((128, D), jnp.float32),
            pltpu.VMEM_SHARED((N_ACC, D), jnp.float32),
            pltpu.SemaphoreType.DMA,
            pltpu.SemaphoreType.DMA,
        ],
    )
    def _sc_scatter(em_hbm, ea_hbm, sidx_hbm, z_hbm,
                    outm_hbm, outa_hbm, sidx_v, mbuf, mbuf2, acc, sem, sem2):
        cid = lax.axis_index("c")
        sid = lax.axis_index("s")
        row0 = sid * ROWS_PER_TILE

        pltpu.sync_copy(z_hbm, mbuf)

        def zbody(t, _):
            pltpu.sync_copy(mbuf, acc.at[pl.ds(row0 + t * 128, 128)])
            return 0

        lax.fori_loop(0, ROWS_PER_TILE // 128, zbody, 0)
        pltpu.sync_copy(sidx_hbm.at[sid], sidx_v)
        plsc.subcore_barrier()

        base = sid * (TCHUNKS * 128)

        def scatter_all(data_hbm):
            pltpu.async_copy(data_hbm.at[pl.ds(base, 128)], mbuf, sem)

            def body(g, _):
                for b in range(2):
                    j = 2 * g + b
                    buf, sm = (mbuf, sem) if b == 0 else (mbuf2, sem2)
                    nbuf, nsm = (mbuf2, sem2) if b == 0 else (mbuf, sem)
                    pltpu.make_async_copy(
                        data_hbm.at[pl.ds(0, 128)], buf, sm).wait()

                    @pl.when(j + 1 < TCHUNKS)
                    def _():
                        pltpu.async_copy(
                            data_hbm.at[pl.ds(base + (j + 1) * 128, 128)],
                            nbuf, nsm)

                    pltpu.sync_copy(buf, acc.at[sidx_v.at[j]], add=True)
                return 0

            lax.fori_loop(0, TCHUNKS // 2, body, 0)

        @pl.when(cid == 0)
        def _():
            scatter_all(em_hbm)

        @pl.when(cid == 1)
        def _():
            scatter_all(ea_hbm)

        plsc.subcore_barrier()

        def copy_out(out_hbm):
            def obody(t, _):
                pltpu.sync_copy(acc.at[pl.ds(row0 + t * 128, 128)], mbuf)
                pltpu.sync_copy(mbuf, out_hbm.at[pl.ds(row0 + t * 128, 128)])
                return 0
            lax.fori_loop(0, ROWS_PER_TILE // 128, obody, 0)

        @pl.when(cid == 0)
        def _():
            copy_out(outm_hbm)

        @pl.when(cid == 1)
        def _():
            copy_out(outa_hbm)

    return _sc_gather, _sc_scatter


def _edge_body(xs_ref, xr_ref, w1s, w1r, w1ps, w1pr, w1a,
               wy1s, wy1r, w2m, w2p, w2a, wy2, outm_ref, outa_ref):
    f32 = jnp.float32
    xs = xs_ref[...]
    xr = xr_ref[...]
    r = xs[:, :3] - xr[:, :3]
    d = jnp.sqrt(jnp.sum(r * r, axis=-1, keepdims=True))
    rh = (r / (d + 1e-8)) * _SR3
    a1, a2, a3 = rh[:, 0:1], rh[:, 1:2], rh[:, 2:3]

    def dot(x, w):
        return jax.lax.dot_general(x.astype(jnp.bfloat16), w[...],
                                   (((1,), (0,)), ((), ())),
                                   preferred_element_type=f32)

    y1s = wy1s[0:1] + a1 * wy1s[1:2] + a2 * wy1s[2:3] + a3 * wy1s[3:4]
    y1r = wy1r[0:1] + a1 * wy1r[1:2] + a2 * wy1r[2:3] + a3 * wy1r[3:4]
    pre = (dot(xs, w1s) + dot(xr, w1r) + dot(xs * y1s, w1ps) + dot(xr * y1r, w1pr)
           + w1a[0:1] + a1 * w1a[1:2] + a2 * w1a[2:3] + a3 * w1a[3:4])
    m = pre[:, 128:] * jax.nn.sigmoid(pre[:, :128])
    y2 = wy2[0:1] + a1 * wy2[1:2] + a2 * wy2[2:3] + a3 * wy2[3:4]
    pre2 = (dot(m, w2m) + dot(m * y2, w2p)
            + w2a[0:1] + a1 * w2a[1:2] + a2 * w2a[2:3] + a3 * w2a[3:4])
    m2 = pre2[:, 128:] * jax.nn.sigmoid(pre2[:, :128])
    ones = jnp.ones_like(a1)
    zeros = jnp.zeros((xs.shape[0], D - 4), f32)
    outm_ref[...] = m2
    outa_ref[...] = jnp.concatenate([a1, a2, a3, ones, zeros], axis=-1)


def _edge_call(gathered, w):
    nblk = E_PAD // EB
    full = lambda arr: pl.BlockSpec(arr.shape, lambda i: (0,) * arr.ndim)
    return pl.pallas_call(
        _edge_body,
        grid=(nblk,),
        in_specs=[pl.BlockSpec((EB, D), lambda i: (i, 0)),
                  pl.BlockSpec((EB, D), lambda i: (i + nblk, 0))]
                 + [full(a) for a in w],
        out_specs=[pl.BlockSpec((EB, D), lambda i: (i, 0)),
                   pl.BlockSpec((EB, D), lambda i: (i, 0))],
        out_shape=[jax.ShapeDtypeStruct((E_PAD, D), jnp.float32),
                   jax.ShapeDtypeStruct((E_PAD, D), jnp.float32)],
    )(gathered, gathered, *w)


def _node_body(x_ref, pm_ref, pa_ref,
               wy1m, wy1a, wn1x, wn1m, wn1p, wn1a, b1,
               wy2m, wy2a, wn2x, wn2m, wn2p, wn2a, b2,
               wlin, blin, out_ref):
    f32 = jnp.float32
    x = x_ref[...]
    msum = pm_ref[...]
    asum = pa_ref[...]
    deg = asum[:, 3:4]
    inv = 1.0 / jnp.maximum(deg, 1.0)
    m_i = msum * inv
    ai0 = deg * inv
    ai1 = asum[:, 0:1] * inv
    ai2 = asum[:, 1:2] * inv
    ai3 = asum[:, 2:3] * inv

    def dot(a, w):
        return jax.lax.dot_general(a.astype(jnp.bfloat16), w[...],
                                   (((1,), (0,)), ((), ())),
                                   preferred_element_type=f32)

    def apart(wa):
        return (ai0 * wa[0:1] + ai1 * wa[1:2] + ai2 * wa[2:3] + ai3 * wa[3:4])

    y1 = dot(m_i, wy1m) + apart(wy1a)
    p1_ = (dot(x, wn1x) + dot(m_i, wn1m) + dot(x * y1, wn1p)
           + apart(wn1a) + b1[0:1])
    x1 = p1_[:, 128:] * jax.nn.sigmoid(p1_[:, :128])
    y2 = dot(m_i, wy2m) + apart(wy2a)
    p2_ = (dot(x1, wn2x) + dot(m_i, wn2m) + dot(x1 * y2, wn2p)
           + apart(wn2a) + b2[0:1])
    x2 = p2_[:, 128:] * jax.nn.sigmoid(p2_[:, :128])
    out_ref[...] = dot(x2, wlin) + blin[0:1]


def _node_call(nodes, parts_m, parts_a, w):
    nblk = N // NB
    full = lambda arr: pl.BlockSpec(arr.shape, lambda i: (0,) * arr.ndim)
    return pl.pallas_call(
        _node_body,
        grid=(nblk,),
        in_specs=[pl.BlockSpec((NB, D), lambda i: (i, 0)),
                  pl.BlockSpec((NB, D), lambda i: (i, 0)),
                  pl.BlockSpec((NB, D), lambda i: (i, 0))]
                 + [full(a) for a in w],
        out_specs=pl.BlockSpec((NB, D), lambda i: (i, 0)),
        out_shape=jax.ShapeDtypeStruct((N, D), jnp.float32),
    )(nodes, parts_m, parts_a, *w)


def _pad8(w):
    return jnp.concatenate([w, jnp.zeros((8 - w.shape[0],) + w.shape[1:], w.dtype)])


def kernel(x, edge_index, Wy_e1, W_e1, b_e1, Wy_e2, W_e2, b_e2,
           Wy_n1, W_n1, b_n1, Wy_n2, W_n2, b_n2, W_lin, b_lin):
    senders = edge_index[0].astype(jnp.int32)
    receivers = edge_index[1].astype(jnp.int32)
    pad = E_PAD - E
    spad = jnp.concatenate([senders, jnp.zeros((pad,), jnp.int32)])
    rpad = jnp.concatenate([receivers, jnp.zeros((pad,), jnp.int32)])
    gidx = jnp.concatenate([spad, rpad]).reshape(GTOT, 128)
    sidx = jnp.concatenate(
        [receivers, jnp.full((pad,), N, jnp.int32)]).reshape(NS, TCHUNKS, 128)
    zeros_m = jnp.zeros((128, D), jnp.float32)

    nodes = x
    bf16 = jnp.bfloat16
    for s in range(2):
        W1 = W_e1[s]
        w1a = _pad8(W1[256:260].at[0].add(b_e1[s]))
        W2 = W_e2[s]
        w2a = _pad8(W2[128:132].at[0].add(b_e2[s]))
        w_edge = (W1[:128].astype(bf16), W1[128:256].astype(bf16),
                  W1[260:388].astype(bf16), W1[388:516].astype(bf16), w1a,
                  _pad8(Wy_e1[s][:, :128]), _pad8(Wy_e1[s][:, 128:]),
                  W2[:128].astype(bf16), W2[132:260].astype(bf16), w2a,
                  _pad8(Wy_e2[s]))
        w_node = (Wy_n1[s][:128].astype(bf16), _pad8(Wy_n1[s][128:132]),
                  W_n1[s][:128].astype(bf16), W_n1[s][128:256].astype(bf16),
                  W_n1[s][260:388].astype(bf16),
                  _pad8(W_n1[s][256:260]), b_n1[s][None, :],
                  Wy_n2[s][:128].astype(bf16), _pad8(Wy_n2[s][128:132]),
                  W_n2[s][:128].astype(bf16), W_n2[s][128:256].astype(bf16),
                  W_n2[s][260:388].astype(bf16),
                  _pad8(W_n2[s][256:260]), b_n2[s][None, :],
                  W_lin[s].astype(bf16), b_lin[s][None, :])

        sc_gather, sc_scatter = _sc_kernels()
        gathered = sc_gather(nodes, gidx)
        edata_m, edata_a = _edge_call(gathered, w_edge)
        parts_m, parts_a = sc_scatter(edata_m, edata_a, sidx, zeros_m)
        nodes = _node_call(nodes, parts_m, parts_a, w_node)
    return nodes
